# Initial kernel scaffold; baseline (speedup 1.0000x reference)
import jax
import jax.numpy as jnp
from jax.experimental import pallas as pl


def kernel(rays_o, rays_d, viewdirs, bg, Wd1, bd1, Wd2, bd2, Wg1, bg1, Wg2, bg2, We1, be1, We2, be2):
    raise NotImplementedError("write your pallas kernel here")


# speedup vs baseline: 8.7961x; 8.7961x over previous
import functools

import jax
import jax.numpy as jnp
from jax.experimental import pallas as pl
from jax.experimental.pallas import tpu as pltpu

N_STEPS = 128
NEAR = 0.2
STEPSIZE = 0.5
VOXEL_SIZE = 0.01
VOXEL_SIZE_RATIO = 1.0
ACT_SHIFT = -4.0
XYZ_MIN = -1.0
XYZ_MAX = 1.0
FAST_THRES = 1e-4
INTERVAL = STEPSIZE * VOXEL_SIZE_RATIO
STEPDIST = STEPSIZE * VOXEL_SIZE

E = 8
H = 64
GH = 64
NH = H + GH + E * H
NHA = NH + 8
NOUT = 48

RB = 32


def _softplus(x):
    return jnp.where(x > 20.0, x, jnp.log1p(jnp.exp(jnp.minimum(x, 20.0))))


def _raw2alpha(raw):
    return 1.0 - jnp.exp(-_softplus(raw + ACT_SHIFT) * INTERVAL)


def _cumprod_lanes(x):
    lane = jax.lax.broadcasted_iota(jnp.int32, x.shape, 1)
    k = 1
    while k < N_STEPS:
        sh = pltpu.roll(x, k, axis=1)
        x = x * jnp.where(lane < k, 1.0, sh)
        k *= 2
    return x


def _shift1_fill1(x):
    lane = jax.lax.broadcasted_iota(jnp.int32, x.shape, 1)
    return jnp.where(lane < 1, 1.0, pltpu.roll(x, 1, axis=1))


def _body(a_ref, b_ref, w1_ref, w2_ref, out_ref):
    nb = RB * N_STEPS
    a = a_ref[...]
    b = b_ref[...]
    lane = jax.lax.broadcasted_iota(jnp.int32, (1, nb), 1)
    step = jnp.remainder(lane, N_STEPS).astype(jnp.float32)
    t = NEAR + STEPDIST * (step + 0.5)
    feat = a + b * t

    u = jnp.maximum(
        jax.lax.dot_general(w1_ref[...], feat, (((1,), (0,)), ((), ())),
                            preferred_element_type=jnp.float32), 0.0)
    out = jax.lax.dot_general(w2_ref[...], u, (((1,), (0,)), ((), ())),
                              preferred_element_type=jnp.float32)

    o3 = out.reshape(NOUT, RB, N_STEPS)
    f3 = feat.reshape(8, RB, N_STEPS)
    x, y, z = f3[0], f3[1], f3[2]
    inb = ((x >= XYZ_MIN) & (x <= XYZ_MAX) & (y >= XYZ_MIN) & (y <= XYZ_MAX)
           & (z >= XYZ_MIN) & (z <= XYZ_MAX))

    a0 = _raw2alpha(o3[0])
    a0 = jnp.where(inb, a0, 0.0)
    m1 = a0 > FAST_THRES
    a0 = jnp.where(m1, a0, 0.0)
    cp0 = _cumprod_lanes(1.0 - a0)
    w0 = a0 * _shift1_fill1(cp0)
    pmask = jnp.where(m1 & (w0 > FAST_THRES), 1.0, 0.0)

    logits = [o3[1 + e] for e in range(E)]
    mx1 = logits[0]
    for e in range(1, E):
        mx1 = jnp.maximum(mx1, logits[e])
    e1 = jnp.zeros_like(mx1)
    for e in range(E - 1, -1, -1):
        e1 = jnp.where(logits[e] == mx1, float(e), e1)
    l2 = [jnp.where(e1 == float(e), -1e30, logits[e]) for e in range(E)]
    mx2 = l2[0]
    for e in range(1, E):
        mx2 = jnp.maximum(mx2, l2[e])
    e2 = jnp.zeros_like(mx2)
    for e in range(E - 1, -1, -1):
        e2 = jnp.where(l2[e] == mx2, float(e), e2)
    g1 = jax.nn.sigmoid(mx1 - mx2)
    g2 = 1.0 - g1

    sels = []
    for c in range(4):
        s1 = jnp.zeros_like(mx1)
        s2 = jnp.zeros_like(mx1)
        for e in range(E):
            plane = o3[9 + 8 * c + e]
            s1 = jnp.where(e1 == float(e), plane, s1)
            s2 = jnp.where(e2 == float(e), plane, s2)
        sels.append((s1, s2))
    rgb = [pmask * (g1 * jax.nn.sigmoid(sels[c][0])
                    + g2 * jax.nn.sigmoid(sels[c][1])) for c in range(3)]
    alpha = pmask * (g1 * _raw2alpha(sels[3][0]) + g2 * _raw2alpha(sels[3][1]))

    cp = _cumprod_lanes(1.0 - alpha)
    w = alpha * _shift1_fill1(cp)
    ail = cp[:, N_STEPS - 1:N_STEPS]
    cols = [jnp.sum(w * rgb[c], axis=1, keepdims=True) for c in range(3)]
    out_ref[...] = jnp.concatenate(cols + [ail], axis=1)


@functools.partial(jax.jit, static_argnames=())
def kernel(rays_o, rays_d, viewdirs, bg, Wd1, bd1, Wd2, bd2,
           Wg1, bg1, Wg2, bg2, We1, be1, We2, be2):
    n_rays = rays_o.shape[0]
    p = n_rays * N_STEPS
    f32 = jnp.float32

    dirs = rays_d / (jnp.linalg.norm(rays_d, axis=-1, keepdims=True) + 1e-8)
    a_rows = jnp.concatenate(
        [rays_o, viewdirs, jnp.zeros((n_rays, 1), f32),
         jnp.ones((n_rays, 1), f32)], axis=1)
    b_rows = jnp.concatenate([dirs, jnp.zeros((n_rays, 5), f32)], axis=1)
    a_t = jnp.repeat(a_rows, N_STEPS, axis=0).T
    b_t = jnp.repeat(b_rows, N_STEPS, axis=0).T

    z = jnp.zeros
    r0 = jnp.concatenate([Wd1.T, z((H, 4), f32), bd1[:, None]], axis=1)
    r1 = jnp.concatenate([Wg1.T, z((GH, 1), f32), bg1[:, None]], axis=1)
    we1r = jnp.transpose(We1, (0, 2, 1)).reshape(E * H, 6)
    r2 = jnp.concatenate([we1r, z((E * H, 1), f32),
                          be1.reshape(E * H, 1)], axis=1)
    r3 = jnp.concatenate([z((8, 7), f32), jnp.ones((8, 1), f32)], axis=1)
    w1 = jnp.concatenate([r0, r1, r2, r3], axis=0)

    row_d = jnp.concatenate([Wd2.T, z((1, GH + E * H), f32),
                             bd2.reshape(1, 1), z((1, 7), f32)], axis=1)
    rows_g = jnp.concatenate([z((E, H), f32), Wg2.T, z((E, E * H), f32),
                              bg2[:, None], z((E, 7), f32)], axis=1)
    we2r = jnp.transpose(We2, (2, 0, 1))
    blk = we2r[:, :, None, :] * jnp.eye(E, dtype=f32)[None, :, :, None]
    w2exp = blk.reshape(4 * E, E * H)
    rows_e = jnp.concatenate([z((4 * E, H + GH), f32), w2exp,
                              be2.T.reshape(4 * E, 1), z((4 * E, 7), f32)],
                             axis=1)
    rows_pad = z((NOUT - 1 - E - 4 * E, NHA), f32)
    w2 = jnp.concatenate([row_d, rows_g, rows_e, rows_pad], axis=0)

    grid = n_rays // RB
    nb = RB * N_STEPS
    res = pl.pallas_call(
        _body,
        grid=(grid,),
        in_specs=[
            pl.BlockSpec((8, nb), lambda i: (0, i)),
            pl.BlockSpec((8, nb), lambda i: (0, i)),
            pl.BlockSpec((NHA, 8), lambda i: (0, 0)),
            pl.BlockSpec((NOUT, NHA), lambda i: (0, 0)),
        ],
        out_specs=pl.BlockSpec((RB, 4), lambda i: (i, 0)),
        out_shape=jax.ShapeDtypeStruct((n_rays, 4), f32),
    )(a_t, b_t, w1, w2)
    return res[:, :3] + res[:, 3:4] * bg[None, :]


# speedup vs baseline: 9.4177x; 1.0707x over previous
import functools

import numpy as np

import jax
import jax.numpy as jnp
from jax.experimental import pallas as pl
from jax.experimental.pallas import tpu as pltpu

N_STEPS = 128
NEAR = 0.2
STEPSIZE = 0.5
VOXEL_SIZE = 0.01
VOXEL_SIZE_RATIO = 1.0
ACT_SHIFT = -4.0
XYZ_MIN = -1.0
XYZ_MAX = 1.0
FAST_THRES = 1e-4
INTERVAL = STEPSIZE * VOXEL_SIZE_RATIO
STEPDIST = STEPSIZE * VOXEL_SIZE

E = 8
H = 64
GH = 64

RB = 32
NB = RB * N_STEPS

ND = H + 8
NR = GH + E * H + 8
NOUTR = 8 + 4 * E

_p = np.arange(NB)
_sel = (_p[None, :] // N_STEPS == np.arange(RB)[:, None]).astype(np.float32)
_t = (NEAR + STEPDIST * ((_p % N_STEPS) + 0.5)).astype(np.float32)
_SS = np.concatenate([_sel, _sel * _t[None, :]], axis=0)


def _softplus(x):
    return jnp.where(x > 20.0, x, jnp.log1p(jnp.exp(jnp.minimum(x, 20.0))))


def _raw2alpha(raw):
    return 1.0 - jnp.exp(-_softplus(raw + ACT_SHIFT) * INTERVAL)


def _cumprod_lanes(x):
    lane = jax.lax.broadcasted_iota(jnp.int32, x.shape, 1)
    k = 1
    while k < N_STEPS:
        sh = pltpu.roll(x, k, axis=1)
        x = x * jnp.where(lane < k, 1.0, sh)
        k *= 2
    return x


def _shift1_fill1(x):
    lane = jax.lax.broadcasted_iota(jnp.int32, x.shape, 1)
    return jnp.where(lane < 1, 1.0, pltpu.roll(x, 1, axis=1))


def _dot(a, b, prec):
    return jax.lax.dot_general(a, b, (((1,), (0,)), ((), ())),
                               preferred_element_type=prec)


def _body(a_ref, b_ref, ss_ref, w1d_ref, w2d_ref, w1r_ref, w2r_ref, out_ref):
    ab = jnp.concatenate([a_ref[0], b_ref[0]], axis=1)
    feat = _dot(ab, ss_ref[...], jnp.float32)

    f3 = feat.reshape(8, RB, N_STEPS)
    x, y, z = f3[0], f3[1], f3[2]
    inb = ((x >= XYZ_MIN) & (x <= XYZ_MAX) & (y >= XYZ_MIN) & (y <= XYZ_MAX)
           & (z >= XYZ_MIN) & (z <= XYZ_MAX))

    ud = jnp.maximum(_dot(w1d_ref[...], feat, jnp.float32), 0.0)
    densrow = _dot(w2d_ref[...], ud, jnp.float32)
    dens = densrow.reshape(8, RB, N_STEPS)[0]
    a0 = _raw2alpha(dens)
    a0 = jnp.where(inb, a0, 0.0)
    m1 = a0 > FAST_THRES
    a0 = jnp.where(m1, a0, 0.0)
    cp0 = _cumprod_lanes(1.0 - a0)
    w0 = a0 * _shift1_fill1(cp0)
    pmask = jnp.where(m1 & (w0 > FAST_THRES), 1.0, 0.0)

    featb = feat.astype(jnp.bfloat16)
    ur = jnp.maximum(_dot(w1r_ref[...], featb, jnp.float32),
                     0.0).astype(jnp.bfloat16)
    outr = _dot(w2r_ref[...], ur, jnp.float32)
    o3 = outr.reshape(NOUTR, RB, N_STEPS)

    logits = [o3[e] for e in range(E)]
    mx1 = logits[0]
    for e in range(1, E):
        mx1 = jnp.maximum(mx1, logits[e])
    e1 = jnp.zeros_like(mx1)
    for e in range(E - 1, -1, -1):
        e1 = jnp.where(logits[e] == mx1, float(e), e1)
    l2 = [jnp.where(e1 == float(e), -1e30, logits[e]) for e in range(E)]
    mx2 = l2[0]
    for e in range(1, E):
        mx2 = jnp.maximum(mx2, l2[e])
    e2 = jnp.zeros_like(mx2)
    for e in range(E - 1, -1, -1):
        e2 = jnp.where(l2[e] == mx2, float(e), e2)
    g1 = jax.nn.sigmoid(mx1 - mx2)
    g2 = 1.0 - g1

    sels = []
    for c in range(4):
        s1 = jnp.zeros_like(mx1)
        s2 = jnp.zeros_like(mx1)
        for e in range(E):
            plane = o3[8 + 8 * c + e]
            s1 = jnp.where(e1 == float(e), plane, s1)
            s2 = jnp.where(e2 == float(e), plane, s2)
        sels.append((s1, s2))
    rgb = [pmask * (g1 * jax.nn.sigmoid(sels[c][0])
                    + g2 * jax.nn.sigmoid(sels[c][1])) for c in range(3)]
    alpha = pmask * (g1 * _raw2alpha(sels[3][0]) + g2 * _raw2alpha(sels[3][1]))

    cp = _cumprod_lanes(1.0 - alpha)
    w = alpha * _shift1_fill1(cp)
    ail = cp[:, N_STEPS - 1:N_STEPS]
    cols = [jnp.sum(w * rgb[c], axis=1, keepdims=True) for c in range(3)]
    out_ref[...] = jnp.concatenate(cols + [ail], axis=1)


@functools.partial(jax.jit, static_argnames=())
def kernel(rays_o, rays_d, viewdirs, bg, Wd1, bd1, Wd2, bd2,
           Wg1, bg1, Wg2, bg2, We1, be1, We2, be2):
    n_rays = rays_o.shape[0]
    nblk = n_rays // RB
    f32 = jnp.float32
    z = jnp.zeros

    dirs = rays_d / (jnp.linalg.norm(rays_d, axis=-1, keepdims=True) + 1e-8)
    a_rows = jnp.concatenate(
        [rays_o, viewdirs, z((n_rays, 1), f32),
         jnp.ones((n_rays, 1), f32)], axis=1)
    b_rows = jnp.concatenate([dirs, z((n_rays, 5), f32)], axis=1)
    a_bl = a_rows.reshape(nblk, RB, 8).transpose(0, 2, 1)
    b_bl = b_rows.reshape(nblk, RB, 8).transpose(0, 2, 1)

    ss = jnp.asarray(_SS)

    w1d = jnp.concatenate([
        jnp.concatenate([Wd1.T, z((H, 4), f32), bd1[:, None]], axis=1),
        jnp.concatenate([z((8, 7), f32), jnp.ones((8, 1), f32)], axis=1),
    ], axis=0)
    w2d = jnp.concatenate([
        jnp.concatenate([Wd2.T, bd2.reshape(1, 1), z((1, 7), f32)], axis=1),
        z((7, ND), f32),
    ], axis=0)

    we1r = jnp.transpose(We1, (0, 2, 1)).reshape(E * H, 6)
    w1r = jnp.concatenate([
        jnp.concatenate([Wg1.T, z((GH, 1), f32), bg1[:, None]], axis=1),
        jnp.concatenate([we1r, z((E * H, 1), f32),
                         be1.reshape(E * H, 1)], axis=1),
        jnp.concatenate([z((8, 7), f32), jnp.ones((8, 1), f32)], axis=1),
    ], axis=0).astype(jnp.bfloat16)
    we2r = jnp.transpose(We2, (2, 0, 1))
    blk = we2r[:, :, None, :] * jnp.eye(E, dtype=f32)[None, :, :, None]
    w2exp = blk.reshape(4 * E, E * H)
    w2r = jnp.concatenate([
        jnp.concatenate([Wg2.T, z((E, E * H), f32), bg2[:, None],
                         z((E, 7), f32)], axis=1),
        jnp.concatenate([z((4 * E, GH), f32), w2exp,
                         be2.T.reshape(4 * E, 1), z((4 * E, 7), f32)],
                        axis=1),
    ], axis=0).astype(jnp.bfloat16)

    res = pl.pallas_call(
        _body,
        grid=(nblk,),
        in_specs=[
            pl.BlockSpec((1, 8, RB), lambda i: (i, 0, 0)),
            pl.BlockSpec((1, 8, RB), lambda i: (i, 0, 0)),
            pl.BlockSpec((2 * RB, NB), lambda i: (0, 0)),
            pl.BlockSpec((ND, 8), lambda i: (0, 0)),
            pl.BlockSpec((8, ND), lambda i: (0, 0)),
            pl.BlockSpec((NR, 8), lambda i: (0, 0)),
            pl.BlockSpec((NOUTR, NR), lambda i: (0, 0)),
        ],
        out_specs=pl.BlockSpec((RB, 4), lambda i: (i, 0)),
        out_shape=jax.ShapeDtypeStruct((n_rays, 4), f32),
    )(a_bl, b_bl, ss, w1d, w2d, w1r, w2r)
    return res[:, :3] + res[:, 3:4] * bg[None, :]
